# baseline (speedup 1.0000x reference)
import functools

import jax
import jax.numpy as jnp
from jax import lax
from jax.experimental import pallas as pl
from jax.experimental.pallas import tpu as pltpu
from jax.experimental.pallas import tpu_sc as plsc

VOCAB = 100000
EMBED = 128
CTX = 4096
B = 4
T = 4096

_info = plsc.get_sparse_core_info()
NC, NS, L = _info.num_cores, _info.num_subcores, _info.num_lanes
NW = NC * NS
TBLK = T // NW
HALF = 2
CROWS = TBLK // HALF
NCH = B * HALF
NBUF = 3


def _body(x_hbm, tok_hbm, pos_hbm, out_hbm,
          idx_v, tok_v, pos_v, spm,
          sem_p, sem_i0, sem_i1, sem_i2, sem_i3,
          sem_i4, sem_i5, sem_i6, sem_i7,
          sem_g0, sem_g1, sem_g2,
          sem_s0, sem_s1, sem_s2,
          sem_o0, sem_o1, sem_o2):
    sems_i = (sem_i0, sem_i1, sem_i2, sem_i3,
              sem_i4, sem_i5, sem_i6, sem_i7)
    sems_g = (sem_g0, sem_g1, sem_g2)
    sems_s = (sem_s0, sem_s1, sem_s2)
    sems_o = (sem_o0, sem_o1, sem_o2)
    cid = lax.axis_index("c")
    sid = lax.axis_index("s")
    wid = sid * NC + cid
    t0 = wid * TBLK

    def bh(c):
        return c // HALF, (c % HALF) * CROWS

    p_desc = pltpu.async_copy(pos_hbm.at[pl.ds(t0, TBLK)], pos_v, sem_p)
    i_descs = []
    for c in range(NCH):
        b, off = bh(c)
        i_descs.append(pltpu.async_copy(
            x_hbm.at[b, pl.ds(t0 + off, CROWS)], idx_v.at[c], sems_i[c]))

    def start_gather(c):
        return pltpu.async_copy(tok_hbm.at[idx_v.at[c]],
                                tok_v.at[c % NBUF], sems_g[c % NBUF])

    def start_out(c):
        b, off = bh(c)
        return pltpu.async_copy(spm.at[sid, c % NBUF],
                                out_hbm.at[b, pl.ds(t0 + off, CROWS)],
                                sems_o[c % NBUF])

    g = [None] * NCH
    s = [None] * NCH
    o = [None] * NCH
    for c in range(NBUF - 1):
        i_descs[c].wait()
        g[c] = start_gather(c)
    for c in range(NCH):
        buf = c % NBUF
        if c >= 1:
            s[c - 1].wait()
            o[c - 1] = start_out(c - 1)
        n = c + NBUF - 1
        if n < NCH:
            i_descs[n].wait()
            g[n] = start_gather(n)
        g[c].wait()
        if c == 0:
            p_desc.wait()

        b, off = bh(c)

        @plsc.parallel_loop(0, CROWS, unroll=2)
        def add_row(r):
            for j in range(EMBED // L):
                d = pl.ds(j * L, L)
                plsc.addupdate(tok_v.at[buf, r, d], pos_v[off + r, d])

        if c >= NBUF:
            o[c - NBUF].wait()
        s[c] = pltpu.async_copy(tok_v.at[buf], spm.at[sid, buf],
                                sems_s[buf])
    s[NCH - 1].wait()
    o[NCH - 1] = start_out(NCH - 1)
    for c in range(NCH - NBUF, NCH):
        o[c].wait()


_mesh = plsc.VectorSubcoreMesh(core_axis_name="c", subcore_axis_name="s")

_sc_call = functools.partial(
    pl.kernel,
    out_type=jax.ShapeDtypeStruct((B, T, EMBED), jnp.float32),
    mesh=_mesh,
    scratch_types=[
        pltpu.VMEM((NCH, CROWS), jnp.int32),
        pltpu.VMEM((NBUF, CROWS, EMBED), jnp.float32),
        pltpu.VMEM((TBLK, EMBED), jnp.float32),
        pltpu.VMEM_SHARED((NS, NBUF, CROWS, EMBED), jnp.float32),
    ] + [pltpu.SemaphoreType.DMA] * 18,
)(_body)


def kernel(x, tok_table, pos_table):
    return _sc_call(x.astype(jnp.int32), tok_table, pos_table)


# speedup vs baseline: 1.0392x; 1.0392x over previous
import functools

import jax
import jax.numpy as jnp
from jax import lax
from jax.experimental import pallas as pl
from jax.experimental.pallas import tpu as pltpu
from jax.experimental.pallas import tpu_sc as plsc

VOCAB = 100000
EMBED = 128
CTX = 4096
B = 4
T = 4096

_info = plsc.get_sparse_core_info()
NC, NS, L = _info.num_cores, _info.num_subcores, _info.num_lanes
NW = NC * NS
TBLK = T // NW
NBUF = 2


def _body(x_hbm, tok_hbm, pos_hbm, out_hbm,
          idx_v, tok_v, pos_v,
          sem_i0, sem_i1, sem_i2, sem_i3, sem_p,
          sem_g0, sem_g1, sem_o0, sem_o1):
    sems_i = (sem_i0, sem_i1, sem_i2, sem_i3)
    sems_g = (sem_g0, sem_g1)
    sems_o = (sem_o0, sem_o1)
    wid = lax.axis_index("s") * NC + lax.axis_index("c")
    t0 = wid * TBLK

    i_descs = [
        pltpu.async_copy(x_hbm.at[b, pl.ds(t0, TBLK)], idx_v.at[b],
                         sems_i[b])
        for b in range(B)
    ]
    p_desc = pltpu.async_copy(pos_hbm.at[pl.ds(t0, TBLK)], pos_v, sem_p)

    def start_gather(b):
        return pltpu.async_copy(tok_hbm.at[idx_v.at[b]], tok_v.at[b % NBUF],
                                sems_g[b % NBUF])

    g = [None] * B
    o = [None] * B
    i_descs[0].wait()
    g[0] = start_gather(0)
    for b in range(B):
        buf = b % NBUF
        if b + 1 < B:
            if b >= 1:
                o[b - 1].wait()
            i_descs[b + 1].wait()
            g[b + 1] = start_gather(b + 1)
        g[b].wait()
        if b == 0:
            p_desc.wait()

        @plsc.parallel_loop(0, TBLK, unroll=2)
        def add_row(r):
            for j in range(EMBED // L):
                d = pl.ds(j * L, L)
                plsc.addupdate(tok_v.at[buf, r, d], pos_v[r, d])

        o[b] = pltpu.async_copy(tok_v.at[buf],
                                out_hbm.at[b, pl.ds(t0, TBLK)],
                                sems_o[buf])
    o[B - 2].wait()
    o[B - 1].wait()


_mesh = plsc.VectorSubcoreMesh(core_axis_name="c", subcore_axis_name="s")

_sc_call = functools.partial(
    pl.kernel,
    out_type=jax.ShapeDtypeStruct((B, T, EMBED), jnp.float32),
    mesh=_mesh,
    scratch_types=[
        pltpu.VMEM((B, TBLK), jnp.int32),
        pltpu.VMEM((NBUF, TBLK, EMBED), jnp.float32),
        pltpu.VMEM((TBLK, EMBED), jnp.float32),
    ] + [pltpu.SemaphoreType.DMA] * 9,
)(_body)


def kernel(x, tok_table, pos_table):
    return _sc_call(x.astype(jnp.int32), tok_table, pos_table)


# speedup vs baseline: 1.0693x; 1.0289x over previous
import functools

import jax
import jax.numpy as jnp
from jax import lax
from jax.experimental import pallas as pl
from jax.experimental.pallas import tpu as pltpu
from jax.experimental.pallas import tpu_sc as plsc

VOCAB = 100000
EMBED = 128
CTX = 4096
B = 4
T = 4096

_info = plsc.get_sparse_core_info()
NC, NS, L = _info.num_cores, _info.num_subcores, _info.num_lanes
NW = NC * NS
TBLK = T // NW
SUB = 4
CROWS = TBLK // SUB
NCH = B * SUB


def _body(x_hbm, tok_hbm, pos_hbm, out_hbm,
          idx_v, tok_v, pos_v, sem_p, sem_o, *sems_g):
    wid = lax.axis_index("s") * NC + lax.axis_index("c")
    t0 = wid * TBLK

    i_descs = [
        pltpu.async_copy(x_hbm.at[b, pl.ds(t0, TBLK)], idx_v.at[b],
                         sems_g[NCH + b])
        for b in range(B)
    ]
    p_desc = pltpu.async_copy(pos_hbm.at[pl.ds(t0, TBLK)], pos_v, sem_p)

    g = [None] * NCH
    for k in range(NCH):
        b, off = k // SUB, (k % SUB) * CROWS
        if off == 0:
            i_descs[b].wait()
        g[k] = pltpu.async_copy(
            tok_hbm.at[idx_v.at[b, pl.ds(off, CROWS)]],
            tok_v.at[k], sems_g[k])

    o = [None] * NCH
    for k in range(NCH):
        b, off = k // SUB, (k % SUB) * CROWS
        g[k].wait()
        if k == 0:
            p_desc.wait()

        @plsc.parallel_loop(0, CROWS, unroll=2)
        def add_row(r):
            for j in range(EMBED // L):
                d = pl.ds(j * L, L)
                plsc.addupdate(tok_v.at[k, r, d], pos_v[off + r, d])

        o[k] = pltpu.async_copy(tok_v.at[k],
                                out_hbm.at[b, pl.ds(t0 + off, CROWS)],
                                sem_o)
    for k in range(NCH):
        o[k].wait()


_mesh = plsc.VectorSubcoreMesh(core_axis_name="c", subcore_axis_name="s")

_sc_call = functools.partial(
    pl.kernel,
    out_type=jax.ShapeDtypeStruct((B, T, EMBED), jnp.float32),
    mesh=_mesh,
    scratch_types=[
        pltpu.VMEM((B, TBLK), jnp.int32),
        pltpu.VMEM((NCH, CROWS, EMBED), jnp.float32),
        pltpu.VMEM((TBLK, EMBED), jnp.float32),
        pltpu.SemaphoreType.DMA,
        pltpu.SemaphoreType.DMA,
    ] + [pltpu.SemaphoreType.DMA] * (NCH + B),
)(_body)


def kernel(x, tok_table, pos_table):
    return _sc_call(x.astype(jnp.int32), tok_table, pos_table)


# speedup vs baseline: 1.0705x; 1.0012x over previous
import functools

import jax
import jax.numpy as jnp
from jax import lax
from jax.experimental import pallas as pl
from jax.experimental.pallas import tpu as pltpu
from jax.experimental.pallas import tpu_sc as plsc

VOCAB = 100000
EMBED = 128
CTX = 4096
B = 4
T = 4096

_info = plsc.get_sparse_core_info()
NC, NS, L = _info.num_cores, _info.num_subcores, _info.num_lanes
NW = NC * NS
TBLK = T // NW
SUB = 2
CROWS = TBLK // SUB
NCH = B * SUB


def _body(x_hbm, tok_hbm, pos_hbm, out_hbm,
          idx_v, tok_v, pos_v, sem_p, sem_o, *sems_g):
    wid = lax.axis_index("s") * NC + lax.axis_index("c")
    t0 = wid * TBLK

    i_descs = [
        pltpu.async_copy(x_hbm.at[b, pl.ds(t0, TBLK)], idx_v.at[b],
                         sems_g[NCH + b])
        for b in range(B)
    ]
    p_desc = pltpu.async_copy(pos_hbm.at[pl.ds(t0, TBLK)], pos_v, sem_p)

    g = [None] * NCH
    for k in range(NCH):
        b, off = k // SUB, (k % SUB) * CROWS
        if off == 0:
            i_descs[b].wait()
        g[k] = pltpu.async_copy(
            tok_hbm.at[idx_v.at[b, pl.ds(off, CROWS)]],
            tok_v.at[k], sems_g[k])

    o = [None] * NCH
    for k in range(NCH):
        b, off = k // SUB, (k % SUB) * CROWS
        g[k].wait()
        if k == 0:
            p_desc.wait()

        @plsc.parallel_loop(0, CROWS, unroll=2)
        def add_row(r):
            for j in range(EMBED // L):
                d = pl.ds(j * L, L)
                plsc.addupdate(tok_v.at[k, r, d], pos_v[off + r, d])

        o[k] = pltpu.async_copy(tok_v.at[k],
                                out_hbm.at[b, pl.ds(t0 + off, CROWS)],
                                sem_o)
    for k in range(NCH):
        o[k].wait()


_mesh = plsc.VectorSubcoreMesh(core_axis_name="c", subcore_axis_name="s")

_sc_call = functools.partial(
    pl.kernel,
    out_type=jax.ShapeDtypeStruct((B, T, EMBED), jnp.float32),
    mesh=_mesh,
    scratch_types=[
        pltpu.VMEM((B, TBLK), jnp.int32),
        pltpu.VMEM((NCH, CROWS, EMBED), jnp.float32),
        pltpu.VMEM((TBLK, EMBED), jnp.float32),
        pltpu.SemaphoreType.DMA,
        pltpu.SemaphoreType.DMA,
    ] + [pltpu.SemaphoreType.DMA] * (NCH + B),
)(_body)


def kernel(x, tok_table, pos_table):
    return _sc_call(x.astype(jnp.int32), tok_table, pos_table)
